# baseline (speedup 1.0000x reference)
import functools

import jax
import jax.numpy as jnp
from jax import lax
from jax.experimental import pallas as pl
from jax.experimental.pallas import tpu as pltpu
from jax.experimental.pallas import tpu_sc as plsc

N_TOK = 8192
K_CB = 8192
D = 256
BN = 1024
BK = 2048
KT = K_CB // BK
NT = N_TOK // BN
BIG = 2 ** 30

NC = 2
NS = 16
NW = NC * NS
BPW = N_TOK // NW
LANES = 16


def _argmin_kernel(zt_ref, cb_ref, zsq_ref, esq_ref, idx_ref,
                   minv_ref, mini_ref):
    ki = pl.program_id(1)
    zt = zt_ref[...]
    cb = cb_ref[...]
    z_sq = zsq_ref[...]
    e_sq = esq_ref[...]
    dot = lax.dot_general(cb.astype(jnp.bfloat16), zt.astype(jnp.bfloat16),
                          (((1,), (0,)), ((), ())),
                          preferred_element_type=jnp.float32)
    dist = (z_sq + e_sq) - 2.0 * dot
    tile_min = jnp.min(dist, axis=0, keepdims=True)
    rows = lax.broadcasted_iota(jnp.int32, (BK, BN), 0) + ki * BK
    cand = jnp.where(dist == tile_min, rows, BIG)
    tile_arg = jnp.min(cand, axis=0, keepdims=True)

    @pl.when(ki == 0)
    def _():
        minv_ref[...] = tile_min.astype(jnp.bfloat16)
        mini_ref[...] = tile_arg

    @pl.when(ki > 0)
    def _():
        acc = minv_ref[...].astype(jnp.float32)
        better = tile_min < acc
        minv_ref[...] = jnp.where(better, tile_min,
                                  acc).astype(jnp.bfloat16)
        mini_ref[...] = jnp.where(better, tile_arg, mini_ref[...])

    @pl.when(ki == KT - 1)
    def _():
        idx_ref[...] = mini_ref[...]


_argmin_call = pl.pallas_call(
    _argmin_kernel,
    grid=(NT, KT),
    in_specs=[
        pl.BlockSpec((D, BN), lambda ni, ki: (0, ni)),
        pl.BlockSpec((BK, D), lambda ni, ki: (ki, 0)),
        pl.BlockSpec((1, BN), lambda ni, ki: (0, ni)),
        pl.BlockSpec((BK, 1), lambda ni, ki: (ki, 0)),
    ],
    out_specs=pl.BlockSpec((1, BN), lambda ni, ki: (0, ni)),
    out_shape=jax.ShapeDtypeStruct((1, N_TOK), jnp.int32),
    scratch_shapes=[
        pltpu.VMEM((1, BN), jnp.bfloat16),
        pltpu.VMEM((1, BN), jnp.int32),
    ],
)


def _sc_gather_kernel(cb_hbm, idx_hbm, zq_hbm, idx_v, rows_v, sem):
    wid = lax.axis_index("s") * NC + lax.axis_index("c")
    base = wid * BPW
    pltpu.sync_copy(idx_hbm.at[pl.ds(base, BPW)], idx_v)
    pltpu.async_copy(cb_hbm.at[idx_v], rows_v, sem).wait()
    pltpu.sync_copy(rows_v, zq_hbm.at[pl.ds(base, BPW)])


@functools.lru_cache(maxsize=1)
def _sc_gather_call():
    return pl.kernel(
        _sc_gather_kernel,
        mesh=plsc.VectorSubcoreMesh(core_axis_name="c", subcore_axis_name="s"),
        out_type=jax.ShapeDtypeStruct((N_TOK, D), jnp.float32),
        scratch_types=[
            pltpu.VMEM((BPW,), jnp.int32),
            pltpu.VMEM((BPW, D), jnp.float32),
            pltpu.SemaphoreType.DMA,
        ],
    )


def _epilogue_kernel(z_ref, zq_ref, idx_ref, zqst_ref, loss_ref, perp_ref,
                     acc_ref, cnt_ref):
    i = pl.program_id(0)
    z = z_ref[...]
    zq = zq_ref[...]
    zqst_ref[...] = z + (zq - z)
    d = z - zq
    part = jnp.sum(d * d, axis=(0, 1), keepdims=True)

    idx = idx_ref[...]
    bins = lax.broadcasted_iota(jnp.int32, (BN, 1), 0) + i * BN
    hits = jnp.where(idx == bins, 1.0, 0.0)
    cnt_ref[pl.ds(i * BN, BN), :] = jnp.sum(hits, axis=1, keepdims=True)

    @pl.when(i == 0)
    def _():
        acc_ref[...] = part

    @pl.when(i > 0)
    def _():
        acc_ref[...] = acc_ref[...] + part

    @pl.when(i == NT - 1)
    def _():
        m = acc_ref[...] * jnp.float32(1.0 / (N_TOK * D))
        loss_ref[...] = m + 0.25 * m
        p = cnt_ref[...] * jnp.float32(1.0 / N_TOK)
        plogp = p * jnp.log(p + 1e-12)
        perp_ref[...] = jnp.exp(-jnp.sum(plogp, axis=(0, 1), keepdims=True))


_epilogue_call = pl.pallas_call(
    _epilogue_kernel,
    grid=(NT,),
    in_specs=[
        pl.BlockSpec((BN, D), lambda i: (i, 0)),
        pl.BlockSpec((BN, D), lambda i: (i, 0)),
        pl.BlockSpec((1, N_TOK), lambda i: (0, 0)),
    ],
    out_specs=[
        pl.BlockSpec((BN, D), lambda i: (i, 0)),
        pl.BlockSpec((1, 1), lambda i: (0, 0)),
        pl.BlockSpec((1, 1), lambda i: (0, 0)),
    ],
    out_shape=[
        jax.ShapeDtypeStruct((N_TOK, D), jnp.float32),
        jax.ShapeDtypeStruct((1, 1), jnp.float32),
        jax.ShapeDtypeStruct((1, 1), jnp.float32),
    ],
    scratch_shapes=[
        pltpu.VMEM((1, 1), jnp.float32),
        pltpu.VMEM((K_CB, 1), jnp.float32),
    ],
)


def kernel(z_e, codebook):
    z2d = z_e.reshape(N_TOK, D)
    zt = z2d.T
    z_sq = jnp.sum(z2d ** 2, axis=1).reshape(1, N_TOK)
    e_sq = jnp.sum(codebook ** 2, axis=1).reshape(K_CB, 1)
    idx_row = _argmin_call(zt, codebook, z_sq, e_sq)
    idx_flat = idx_row.reshape(N_TOK)
    zq2d = _sc_gather_call()(codebook, idx_flat)
    zqst2d, loss, perp = _epilogue_call(z2d, zq2d, idx_row)
    return (zqst2d.reshape(z_e.shape),
            idx_flat.reshape(z_e.shape[:-1]),
            loss.reshape(()),
            perp.reshape(()))


# speedup vs baseline: 1.2985x; 1.2985x over previous
import functools

import jax
import jax.numpy as jnp
from jax import lax
from jax.experimental import pallas as pl
from jax.experimental.pallas import tpu as pltpu
from jax.experimental.pallas import tpu_sc as plsc

N_TOK = 8192
K_CB = 8192
D = 256
BN = 1024
BK = 2048
KT = K_CB // BK
NT = N_TOK // BN
BIG = 2 ** 30
CH = 8
NCH = BK // CH

NC = 2
NS = 16
NW = NC * NS
BPW = N_TOK // NW
CNT_W = 16


def _argmin_kernel(zt_ref, cb_ref, zsqh_ref, esqh_ref, idx_ref,
                   minv_ref, mini_ref):
    ki = pl.program_id(1)
    zt = zt_ref[...]
    cb = cb_ref[...]
    dot = lax.dot_general(cb.astype(jnp.bfloat16), zt.astype(jnp.bfloat16),
                          (((1,), (0,)), ((), ())),
                          preferred_element_type=jnp.float32)
    zsqh = jnp.broadcast_to(zsqh_ref[...], (CH, BN))
    acc_v = jnp.full((CH, BN), jnp.inf, jnp.float32)
    acc_c = jnp.zeros((CH, BN), jnp.int32)
    for i in range(NCH):
        d = dot[CH * i:CH * i + CH, :]
        s = zsqh + esqh_ref[pl.ds(CH * i, CH), :]
        dist = s - d
        upd = dist < acc_v
        acc_v = jnp.where(upd, dist, acc_v)
        acc_c = jnp.where(upd, i, acc_c)
    tile_min = jnp.min(acc_v, axis=0, keepdims=True)
    rows = (acc_c * CH + lax.broadcasted_iota(jnp.int32, (CH, BN), 0)
            + ki * BK)
    cand = jnp.where(acc_v == tile_min, rows, BIG)
    tile_arg = jnp.min(cand, axis=0, keepdims=True)

    @pl.when(ki == 0)
    def _():
        minv_ref[...] = tile_min.astype(jnp.bfloat16)
        mini_ref[...] = tile_arg

    @pl.when(ki > 0)
    def _():
        acc = minv_ref[...].astype(jnp.float32)
        better = tile_min < acc
        minv_ref[...] = jnp.where(better, tile_min,
                                  acc).astype(jnp.bfloat16)
        mini_ref[...] = jnp.where(better, tile_arg, mini_ref[...])

    @pl.when(ki == KT - 1)
    def _():
        idx_ref[...] = mini_ref[...]


_argmin_call = pl.pallas_call(
    _argmin_kernel,
    grid=(NT, KT),
    in_specs=[
        pl.BlockSpec((D, BN), lambda ni, ki: (0, ni)),
        pl.BlockSpec((BK, D), lambda ni, ki: (ki, 0)),
        pl.BlockSpec((1, BN), lambda ni, ki: (0, ni)),
        pl.BlockSpec((BK, 1), lambda ni, ki: (ki, 0)),
    ],
    out_specs=pl.BlockSpec((1, BN), lambda ni, ki: (0, ni)),
    out_shape=jax.ShapeDtypeStruct((1, N_TOK), jnp.int32),
    scratch_shapes=[
        pltpu.VMEM((1, BN), jnp.bfloat16),
        pltpu.VMEM((1, BN), jnp.int32),
    ],
)


def _sc_gather_kernel(cb_hbm, idx_hbm, zq_hbm, idx_v, rows_v, sem):
    wid = lax.axis_index("s") * NC + lax.axis_index("c")
    base = wid * BPW
    pltpu.sync_copy(idx_hbm.at[pl.ds(base, BPW)], idx_v)
    pltpu.async_copy(cb_hbm.at[idx_v], rows_v, sem).wait()
    pltpu.sync_copy(rows_v, zq_hbm.at[pl.ds(base, BPW)])


@functools.lru_cache(maxsize=1)
def _sc_gather_call():
    return pl.kernel(
        _sc_gather_kernel,
        mesh=plsc.VectorSubcoreMesh(core_axis_name="c", subcore_axis_name="s"),
        out_type=jax.ShapeDtypeStruct((N_TOK, D), jnp.float32),
        scratch_types=[
            pltpu.VMEM((BPW,), jnp.int32),
            pltpu.VMEM((BPW, D), jnp.float32),
            pltpu.SemaphoreType.DMA,
        ],
    )


def _epilogue_kernel(z_ref, zq_ref, idx_ref, zqst_ref, loss_ref, perp_ref,
                     acc_ref, cnt_ref):
    i = pl.program_id(0)
    z = z_ref[...]
    zq = zq_ref[...]
    zqst_ref[...] = z + (zq - z)
    d = z - zq
    part = jnp.sum(d * d, axis=(0, 1), keepdims=True)

    idx = idx_ref[...]
    bins = lax.broadcasted_iota(jnp.int32, (BN, 1), 0) + i * BN
    hits = jnp.where(idx == bins, 1.0, 0.0)
    cnt_ref[pl.ds(i * BN, BN), :] = jnp.sum(hits, axis=1, keepdims=True)

    @pl.when(i == 0)
    def _():
        acc_ref[...] = part

    @pl.when(i > 0)
    def _():
        acc_ref[...] = acc_ref[...] + part

    @pl.when(i == NT - 1)
    def _():
        m = acc_ref[...] * jnp.float32(1.0 / (N_TOK * D))
        loss_ref[...] = m + 0.25 * m
        p = cnt_ref[...] * jnp.float32(1.0 / N_TOK)
        plogp = p * jnp.log(p + 1e-12)
        perp_ref[...] = jnp.exp(-jnp.sum(plogp, axis=(0, 1), keepdims=True))


_epilogue_call = pl.pallas_call(
    _epilogue_kernel,
    grid=(NT,),
    in_specs=[
        pl.BlockSpec((BN, D), lambda i: (i, 0)),
        pl.BlockSpec((BN, D), lambda i: (i, 0)),
        pl.BlockSpec((1, N_TOK), lambda i: (0, 0)),
    ],
    out_specs=[
        pl.BlockSpec((BN, D), lambda i: (i, 0)),
        pl.BlockSpec((1, 1), lambda i: (0, 0)),
        pl.BlockSpec((1, 1), lambda i: (0, 0)),
    ],
    out_shape=[
        jax.ShapeDtypeStruct((N_TOK, D), jnp.float32),
        jax.ShapeDtypeStruct((1, 1), jnp.float32),
        jax.ShapeDtypeStruct((1, 1), jnp.float32),
    ],
    scratch_shapes=[
        pltpu.VMEM((1, 1), jnp.float32),
        pltpu.VMEM((K_CB, 1), jnp.float32),
    ],
)


def kernel(z_e, codebook):
    z2d = z_e.reshape(N_TOK, D)
    zt = z2d.T
    z_sqh = (jnp.sum(z2d ** 2, axis=1) * 0.5).reshape(1, N_TOK)
    e_sqh = (jnp.sum(codebook ** 2, axis=1) * 0.5).reshape(K_CB, 1)
    idx_row = _argmin_call(zt, codebook, z_sqh, e_sqh)
    idx_flat = idx_row.reshape(N_TOK)
    zq2d = _sc_gather_call()(codebook, idx_flat)
    zqst2d, loss, perp = _epilogue_call(z2d, zq2d, idx_row)
    return (zqst2d.reshape(z_e.shape),
            idx_flat.reshape(z_e.shape[:-1]),
            loss.reshape(()),
            perp.reshape(()))


# speedup vs baseline: 1.5027x; 1.1573x over previous
import functools

import jax
import jax.numpy as jnp
from jax import lax
from jax.experimental import pallas as pl
from jax.experimental.pallas import tpu as pltpu
from jax.experimental.pallas import tpu_sc as plsc

N_TOK = 8192
K_CB = 8192
D = 256
BN = 1024
BK = 2048
KT = K_CB // BK
NT = N_TOK // BN
BIG = 2 ** 30
CH = 8
NCH = BK // CH

NC = 2
NS = 16
NW = NC * NS
BPW = N_TOK // NW
CNT_W = 16


def _argmin_kernel(zt_ref, cb_ref, zsqh_ref, esqh_ref, idx_ref,
                   minv_ref, mini_ref):
    ki = pl.program_id(1)
    zt = zt_ref[...]
    cb = cb_ref[...]
    dot = lax.dot_general(cb.astype(jnp.bfloat16), zt.astype(jnp.bfloat16),
                          (((1,), (0,)), ((), ())),
                          preferred_element_type=jnp.float32)
    zsqh = jnp.broadcast_to(zsqh_ref[...], (CH, BN))
    acc_v = jnp.full((CH, BN), jnp.inf, jnp.float32)
    acc_c = jnp.zeros((CH, BN), jnp.int32)
    for i in range(NCH):
        d = dot[CH * i:CH * i + CH, :]
        s = zsqh + esqh_ref[pl.ds(CH * i, CH), :]
        dist = s - d
        upd = dist < acc_v
        acc_v = jnp.where(upd, dist, acc_v)
        acc_c = jnp.where(upd, i, acc_c)
    tile_min = jnp.min(acc_v, axis=0, keepdims=True)
    rows = (acc_c * CH + lax.broadcasted_iota(jnp.int32, (CH, BN), 0)
            + ki * BK)
    cand = jnp.where(acc_v == tile_min, rows, BIG)
    tile_arg = jnp.min(cand, axis=0, keepdims=True)

    @pl.when(ki == 0)
    def _():
        minv_ref[...] = tile_min.astype(jnp.bfloat16)
        mini_ref[...] = tile_arg

    @pl.when(ki > 0)
    def _():
        acc = minv_ref[...].astype(jnp.float32)
        better = tile_min < acc
        minv_ref[...] = jnp.where(better, tile_min,
                                  acc).astype(jnp.bfloat16)
        mini_ref[...] = jnp.where(better, tile_arg, mini_ref[...])

    @pl.when(ki == KT - 1)
    def _():
        idx_ref[...] = mini_ref[...]


_argmin_call = pl.pallas_call(
    _argmin_kernel,
    grid=(NT, KT),
    in_specs=[
        pl.BlockSpec((D, BN), lambda ni, ki: (0, ni)),
        pl.BlockSpec((BK, D), lambda ni, ki: (ki, 0)),
        pl.BlockSpec((1, BN), lambda ni, ki: (0, ni)),
        pl.BlockSpec((BK, 1), lambda ni, ki: (ki, 0)),
    ],
    out_specs=pl.BlockSpec((1, BN), lambda ni, ki: (0, ni)),
    out_shape=jax.ShapeDtypeStruct((1, N_TOK), jnp.int32),
    scratch_shapes=[
        pltpu.VMEM((1, BN), jnp.bfloat16),
        pltpu.VMEM((1, BN), jnp.int32),
    ],
)


def _sc_gather_kernel(cb_hbm, idx_hbm, zq_hbm, idx_v, rows_v, sem):
    wid = lax.axis_index("s") * NC + lax.axis_index("c")
    base = wid * BPW
    pltpu.sync_copy(idx_hbm.at[pl.ds(base, BPW)], idx_v)
    pltpu.async_copy(cb_hbm.at[idx_v], rows_v, sem).wait()
    pltpu.sync_copy(rows_v, zq_hbm.at[pl.ds(base, BPW)])


@functools.lru_cache(maxsize=1)
def _sc_gather_call():
    return pl.kernel(
        _sc_gather_kernel,
        mesh=plsc.VectorSubcoreMesh(core_axis_name="c", subcore_axis_name="s"),
        out_type=jax.ShapeDtypeStruct((N_TOK, D), jnp.float32),
        scratch_types=[
            pltpu.VMEM((BPW,), jnp.int32),
            pltpu.VMEM((BPW, D), jnp.float32),
            pltpu.SemaphoreType.DMA,
        ],
    )


def _epilogue_kernel(z_ref, zq_ref, idx_ref, zqst_ref, loss_ref, perp_ref,
                     acc_ref):
    i = pl.program_id(0)
    z = z_ref[...]
    zq = zq_ref[...]
    zqst_ref[...] = z + (zq - z)
    d = z - zq
    part = jnp.sum(d * d, axis=(0, 1), keepdims=True)

    @pl.when(i == 0)
    def _():
        acc_ref[...] = part

    @pl.when(i > 0)
    def _():
        acc_ref[...] = acc_ref[...] + part

    @pl.when(i == NT - 1)
    def _():
        m = acc_ref[...] * jnp.float32(1.0 / (N_TOK * D))
        loss_ref[...] = m + 0.25 * m
        idx = idx_ref[...]
        hi_bins = lax.broadcasted_iota(jnp.int32, (64, 1), 0)
        lo_bins = lax.broadcasted_iota(jnp.int32, (128, 1), 0)
        oh_hi = jnp.where(lax.shift_right_logical(idx, 7) == hi_bins,
                          1.0, 0.0).astype(jnp.bfloat16)
        oh_lo = jnp.where((idx & 127) == lo_bins,
                          1.0, 0.0).astype(jnp.bfloat16)
        counts = lax.dot_general(oh_hi, oh_lo, (((1,), (1,)), ((), ())),
                                 preferred_element_type=jnp.float32)
        p = counts * jnp.float32(1.0 / N_TOK)
        plogp = p * jnp.log(p + 1e-12)
        perp_ref[...] = jnp.exp(-jnp.sum(plogp, axis=(0, 1), keepdims=True))


_epilogue_call = pl.pallas_call(
    _epilogue_kernel,
    grid=(NT,),
    in_specs=[
        pl.BlockSpec((BN, D), lambda i: (i, 0)),
        pl.BlockSpec((BN, D), lambda i: (i, 0)),
        pl.BlockSpec((1, N_TOK), lambda i: (0, 0)),
    ],
    out_specs=[
        pl.BlockSpec((BN, D), lambda i: (i, 0)),
        pl.BlockSpec((1, 1), lambda i: (0, 0)),
        pl.BlockSpec((1, 1), lambda i: (0, 0)),
    ],
    out_shape=[
        jax.ShapeDtypeStruct((N_TOK, D), jnp.float32),
        jax.ShapeDtypeStruct((1, 1), jnp.float32),
        jax.ShapeDtypeStruct((1, 1), jnp.float32),
    ],
    scratch_shapes=[pltpu.VMEM((1, 1), jnp.float32)],
)


def kernel(z_e, codebook):
    z2d = z_e.reshape(N_TOK, D)
    zt = z2d.T
    z_sqh = (jnp.sum(z2d ** 2, axis=1) * 0.5).reshape(1, N_TOK)
    e_sqh = (jnp.sum(codebook ** 2, axis=1) * 0.5).reshape(K_CB, 1)
    idx_row = _argmin_call(zt, codebook, z_sqh, e_sqh)
    idx_flat = idx_row.reshape(N_TOK)
    zq2d = _sc_gather_call()(codebook, idx_flat)
    zqst2d, loss, perp = _epilogue_call(z2d, zq2d, idx_row)
    return (zqst2d.reshape(z_e.shape),
            idx_flat.reshape(z_e.shape[:-1]),
            loss.reshape(()),
            perp.reshape(()))


# speedup vs baseline: 1.5933x; 1.0603x over previous
import functools

import jax
import jax.numpy as jnp
from jax import lax
from jax.experimental import pallas as pl
from jax.experimental.pallas import tpu as pltpu
from jax.experimental.pallas import tpu_sc as plsc

N_TOK = 8192
K_CB = 8192
D = 256
BN = 1024
BK = 2048
KT = K_CB // BK
NT = N_TOK // BN
BIG = 2 ** 30
CH = 8
NCH = BK // CH

NC = 2
NS = 16
NW = NC * NS
BPW = N_TOK // NW
CNT_W = 16


def _argmin_kernel(z_ref, cb_ref, zsqh_ref, esqh_ref, idx_ref,
                   minv_ref, mini_ref, zt_ref):
    ki = pl.program_id(1)

    @pl.when(ki == 0)
    def _():
        zt_ref[...] = z_ref[...].T

    zt = zt_ref[...]
    cb = cb_ref[...]
    dot = lax.dot_general(cb.astype(jnp.bfloat16), zt.astype(jnp.bfloat16),
                          (((1,), (0,)), ((), ())),
                          preferred_element_type=jnp.float32)
    zsqh = jnp.broadcast_to(zsqh_ref[...], (CH, BN))
    acc_v = jnp.full((CH, BN), jnp.inf, jnp.float32)
    acc_c = jnp.zeros((CH, BN), jnp.int32)
    for i in range(NCH):
        d = dot[CH * i:CH * i + CH, :]
        s = zsqh + esqh_ref[pl.ds(CH * i, CH), :]
        dist = s - d
        upd = dist < acc_v
        acc_v = jnp.where(upd, dist, acc_v)
        acc_c = jnp.where(upd, i, acc_c)
    tile_min = jnp.min(acc_v, axis=0, keepdims=True)
    rows = (acc_c * CH + lax.broadcasted_iota(jnp.int32, (CH, BN), 0)
            + ki * BK)
    cand = jnp.where(acc_v == tile_min, rows, BIG)
    tile_arg = jnp.min(cand, axis=0, keepdims=True)

    @pl.when(ki == 0)
    def _():
        minv_ref[...] = tile_min.astype(jnp.bfloat16)
        mini_ref[...] = tile_arg

    @pl.when(ki > 0)
    def _():
        acc = minv_ref[...].astype(jnp.float32)
        better = tile_min < acc
        minv_ref[...] = jnp.where(better, tile_min,
                                  acc).astype(jnp.bfloat16)
        mini_ref[...] = jnp.where(better, tile_arg, mini_ref[...])

    @pl.when(ki == KT - 1)
    def _():
        idx_ref[...] = mini_ref[...]


_argmin_call = pl.pallas_call(
    _argmin_kernel,
    grid=(NT, KT),
    in_specs=[
        pl.BlockSpec((BN, D), lambda ni, ki: (ni, 0)),
        pl.BlockSpec((BK, D), lambda ni, ki: (ki, 0)),
        pl.BlockSpec((1, BN), lambda ni, ki: (0, ni)),
        pl.BlockSpec((BK, 1), lambda ni, ki: (ki, 0)),
    ],
    out_specs=pl.BlockSpec((1, BN), lambda ni, ki: (0, ni)),
    out_shape=jax.ShapeDtypeStruct((1, N_TOK), jnp.int32),
    scratch_shapes=[
        pltpu.VMEM((1, BN), jnp.bfloat16),
        pltpu.VMEM((1, BN), jnp.int32),
        pltpu.VMEM((D, BN), jnp.float32),
    ],
)


def _sc_gather_kernel(cb_hbm, idx_hbm, zq_hbm, idx_v, rows_v, sem):
    wid = lax.axis_index("s") * NC + lax.axis_index("c")
    base = wid * BPW
    pltpu.sync_copy(idx_hbm.at[pl.ds(base, BPW)], idx_v)
    pltpu.async_copy(cb_hbm.at[idx_v], rows_v, sem).wait()
    pltpu.sync_copy(rows_v, zq_hbm.at[pl.ds(base, BPW)])


@functools.lru_cache(maxsize=1)
def _sc_gather_call():
    return pl.kernel(
        _sc_gather_kernel,
        mesh=plsc.VectorSubcoreMesh(core_axis_name="c", subcore_axis_name="s"),
        out_type=jax.ShapeDtypeStruct((N_TOK, D), jnp.float32),
        scratch_types=[
            pltpu.VMEM((BPW,), jnp.int32),
            pltpu.VMEM((BPW, D), jnp.float32),
            pltpu.SemaphoreType.DMA,
        ],
    )


def _epilogue_kernel(z_ref, zq_ref, idx_ref, zqst_ref, loss_ref, perp_ref,
                     acc_ref):
    i = pl.program_id(0)
    z = z_ref[...]
    zq = zq_ref[...]
    zqst_ref[...] = z + (zq - z)
    d = z - zq
    part = jnp.sum(d * d, axis=(0, 1), keepdims=True)

    @pl.when(i == 0)
    def _():
        acc_ref[...] = part

    @pl.when(i > 0)
    def _():
        acc_ref[...] = acc_ref[...] + part

    @pl.when(i == NT - 1)
    def _():
        m = acc_ref[...] * jnp.float32(1.0 / (N_TOK * D))
        loss_ref[...] = m + 0.25 * m
        idx = idx_ref[...]
        hi_bins = lax.broadcasted_iota(jnp.int32, (64, 1), 0)
        lo_bins = lax.broadcasted_iota(jnp.int32, (128, 1), 0)
        oh_hi = jnp.where(lax.shift_right_logical(idx, 7) == hi_bins,
                          1.0, 0.0).astype(jnp.bfloat16)
        oh_lo = jnp.where((idx & 127) == lo_bins,
                          1.0, 0.0).astype(jnp.bfloat16)
        counts = lax.dot_general(oh_hi, oh_lo, (((1,), (1,)), ((), ())),
                                 preferred_element_type=jnp.float32)
        p = counts * jnp.float32(1.0 / N_TOK)
        plogp = p * jnp.log(p + 1e-12)
        perp_ref[...] = jnp.exp(-jnp.sum(plogp, axis=(0, 1), keepdims=True))


_epilogue_call = pl.pallas_call(
    _epilogue_kernel,
    grid=(NT,),
    in_specs=[
        pl.BlockSpec((BN, D), lambda i: (i, 0)),
        pl.BlockSpec((BN, D), lambda i: (i, 0)),
        pl.BlockSpec((1, N_TOK), lambda i: (0, 0)),
    ],
    out_specs=[
        pl.BlockSpec((BN, D), lambda i: (i, 0)),
        pl.BlockSpec((1, 1), lambda i: (0, 0)),
        pl.BlockSpec((1, 1), lambda i: (0, 0)),
    ],
    out_shape=[
        jax.ShapeDtypeStruct((N_TOK, D), jnp.float32),
        jax.ShapeDtypeStruct((1, 1), jnp.float32),
        jax.ShapeDtypeStruct((1, 1), jnp.float32),
    ],
    scratch_shapes=[pltpu.VMEM((1, 1), jnp.float32)],
)


def kernel(z_e, codebook):
    z2d = z_e.reshape(N_TOK, D)
    z_sqh = (jnp.sum(z2d ** 2, axis=1) * 0.5).reshape(1, N_TOK)
    e_sqh = (jnp.sum(codebook ** 2, axis=1) * 0.5).reshape(K_CB, 1)
    idx_row = _argmin_call(z2d, codebook, z_sqh, e_sqh)
    idx_flat = idx_row.reshape(N_TOK)
    zq2d = _sc_gather_call()(codebook, idx_flat)
    zqst2d, loss, perp = _epilogue_call(z2d, zq2d, idx_row)
    return (zqst2d.reshape(z_e.shape),
            idx_flat.reshape(z_e.shape[:-1]),
            loss.reshape(()),
            perp.reshape(()))


# speedup vs baseline: 1.6856x; 1.0580x over previous
import functools

import jax
import jax.numpy as jnp
from jax import lax
from jax.experimental import pallas as pl
from jax.experimental.pallas import tpu as pltpu
from jax.experimental.pallas import tpu_sc as plsc

N_TOK = 8192
K_CB = 8192
D = 256
BN = 2048
BK = 2048
KT = K_CB // BK
NT = N_TOK // BN
BIG = 2 ** 30
CH = 8
NCH = BK // CH

NC = 2
NS = 16
NW = NC * NS
BPW = N_TOK // NW
CNT_W = 16


def _argmin_kernel(z_ref, cb_ref, zsqh_ref, esqh_ref, idx_ref,
                   minv_ref, mini_ref, zt_ref):
    ki = pl.program_id(1)

    @pl.when(ki == 0)
    def _():
        zt_ref[...] = z_ref[...].T

    zt = zt_ref[...]
    cb = cb_ref[...]
    dot = lax.dot_general(cb.astype(jnp.bfloat16), zt.astype(jnp.bfloat16),
                          (((1,), (0,)), ((), ())),
                          preferred_element_type=jnp.float32)
    zsqh = jnp.broadcast_to(zsqh_ref[...], (CH, BN))
    acc_v = jnp.full((CH, BN), jnp.inf, jnp.float32)
    acc_c = jnp.zeros((CH, BN), jnp.int32)
    for i in range(NCH):
        d = dot[CH * i:CH * i + CH, :]
        s = zsqh + esqh_ref[pl.ds(CH * i, CH), :]
        dist = s - d
        upd = dist < acc_v
        acc_v = jnp.where(upd, dist, acc_v)
        acc_c = jnp.where(upd, i, acc_c)
    tile_min = jnp.min(acc_v, axis=0, keepdims=True)
    rows = (acc_c * CH + lax.broadcasted_iota(jnp.int32, (CH, BN), 0)
            + ki * BK)
    cand = jnp.where(acc_v == tile_min, rows, BIG)
    tile_arg = jnp.min(cand, axis=0, keepdims=True)

    @pl.when(ki == 0)
    def _():
        minv_ref[...] = tile_min.astype(jnp.bfloat16)
        mini_ref[...] = tile_arg

    @pl.when(ki > 0)
    def _():
        acc = minv_ref[...].astype(jnp.float32)
        better = tile_min < acc
        minv_ref[...] = jnp.where(better, tile_min,
                                  acc).astype(jnp.bfloat16)
        mini_ref[...] = jnp.where(better, tile_arg, mini_ref[...])

    @pl.when(ki == KT - 1)
    def _():
        idx_ref[...] = mini_ref[...]


_argmin_call = pl.pallas_call(
    _argmin_kernel,
    grid=(NT, KT),
    in_specs=[
        pl.BlockSpec((BN, D), lambda ni, ki: (ni, 0)),
        pl.BlockSpec((BK, D), lambda ni, ki: (ki, 0)),
        pl.BlockSpec((1, BN), lambda ni, ki: (0, ni)),
        pl.BlockSpec((BK, 1), lambda ni, ki: (ki, 0)),
    ],
    out_specs=pl.BlockSpec((1, BN), lambda ni, ki: (0, ni)),
    out_shape=jax.ShapeDtypeStruct((1, N_TOK), jnp.int32),
    scratch_shapes=[
        pltpu.VMEM((1, BN), jnp.bfloat16),
        pltpu.VMEM((1, BN), jnp.int32),
        pltpu.VMEM((D, BN), jnp.float32),
    ],
)


def _sc_gather_kernel(cb_hbm, idx_hbm, zq_hbm, idx_v, rows_v, sem):
    wid = lax.axis_index("s") * NC + lax.axis_index("c")
    base = wid * BPW
    pltpu.sync_copy(idx_hbm.at[pl.ds(base, BPW)], idx_v)
    pltpu.async_copy(cb_hbm.at[idx_v], rows_v, sem).wait()
    pltpu.sync_copy(rows_v, zq_hbm.at[pl.ds(base, BPW)])


@functools.lru_cache(maxsize=1)
def _sc_gather_call():
    return pl.kernel(
        _sc_gather_kernel,
        mesh=plsc.VectorSubcoreMesh(core_axis_name="c", subcore_axis_name="s"),
        out_type=jax.ShapeDtypeStruct((N_TOK, D), jnp.float32),
        scratch_types=[
            pltpu.VMEM((BPW,), jnp.int32),
            pltpu.VMEM((BPW, D), jnp.float32),
            pltpu.SemaphoreType.DMA,
        ],
    )


def _epilogue_kernel(z_ref, zq_ref, idx_ref, zqst_ref, loss_ref, perp_ref,
                     acc_ref):
    i = pl.program_id(0)
    z = z_ref[...]
    zq = zq_ref[...]
    zqst_ref[...] = z + (zq - z)
    d = z - zq
    part = jnp.sum(d * d, axis=(0, 1), keepdims=True)

    @pl.when(i == 0)
    def _():
        acc_ref[...] = part

    @pl.when(i > 0)
    def _():
        acc_ref[...] = acc_ref[...] + part

    @pl.when(i == NT - 1)
    def _():
        m = acc_ref[...] * jnp.float32(1.0 / (N_TOK * D))
        loss_ref[...] = m + 0.25 * m
        idx = idx_ref[...]
        hi_bins = lax.broadcasted_iota(jnp.int32, (64, 1), 0)
        lo_bins = lax.broadcasted_iota(jnp.int32, (128, 1), 0)
        oh_hi = jnp.where(lax.shift_right_logical(idx, 7) == hi_bins,
                          1.0, 0.0).astype(jnp.bfloat16)
        oh_lo = jnp.where((idx & 127) == lo_bins,
                          1.0, 0.0).astype(jnp.bfloat16)
        counts = lax.dot_general(oh_hi, oh_lo, (((1,), (1,)), ((), ())),
                                 preferred_element_type=jnp.float32)
        p = counts * jnp.float32(1.0 / N_TOK)
        plogp = p * jnp.log(p + 1e-12)
        perp_ref[...] = jnp.exp(-jnp.sum(plogp, axis=(0, 1), keepdims=True))


_epilogue_call = pl.pallas_call(
    _epilogue_kernel,
    grid=(NT,),
    in_specs=[
        pl.BlockSpec((BN, D), lambda i: (i, 0)),
        pl.BlockSpec((BN, D), lambda i: (i, 0)),
        pl.BlockSpec((1, N_TOK), lambda i: (0, 0)),
    ],
    out_specs=[
        pl.BlockSpec((BN, D), lambda i: (i, 0)),
        pl.BlockSpec((1, 1), lambda i: (0, 0)),
        pl.BlockSpec((1, 1), lambda i: (0, 0)),
    ],
    out_shape=[
        jax.ShapeDtypeStruct((N_TOK, D), jnp.float32),
        jax.ShapeDtypeStruct((1, 1), jnp.float32),
        jax.ShapeDtypeStruct((1, 1), jnp.float32),
    ],
    scratch_shapes=[pltpu.VMEM((1, 1), jnp.float32)],
)


def kernel(z_e, codebook):
    z2d = z_e.reshape(N_TOK, D)
    z_sqh = (jnp.sum(z2d ** 2, axis=1) * 0.5).reshape(1, N_TOK)
    e_sqh = (jnp.sum(codebook ** 2, axis=1) * 0.5).reshape(K_CB, 1)
    idx_row = _argmin_call(z2d, codebook, z_sqh, e_sqh)
    idx_flat = idx_row.reshape(N_TOK)
    zq2d = _sc_gather_call()(codebook, idx_flat)
    zqst2d, loss, perp = _epilogue_call(z2d, zq2d, idx_row)
    return (zqst2d.reshape(z_e.shape),
            idx_flat.reshape(z_e.shape[:-1]),
            loss.reshape(()),
            perp.reshape(()))


# speedup vs baseline: 1.7100x; 1.0145x over previous
import functools

import jax
import jax.numpy as jnp
from jax import lax
from jax.experimental import pallas as pl
from jax.experimental.pallas import tpu as pltpu
from jax.experimental.pallas import tpu_sc as plsc

N_TOK = 8192
K_CB = 8192
D = 256
BN = 4096
BK = 2048
KT = K_CB // BK
NT = N_TOK // BN
BIG = 2 ** 30
CH = 8
NCH = BK // CH

NC = 2
NS = 16
NW = NC * NS
BPW = N_TOK // NW
CNT_W = 16


def _argmin_kernel(z_ref, cb_ref, zsqh_ref, esqh_ref, idx_ref,
                   minv_ref, mini_ref, zt_ref):
    ki = pl.program_id(1)

    @pl.when(ki == 0)
    def _():
        zt_ref[...] = z_ref[...].T

    zt = zt_ref[...]
    cb = cb_ref[...]
    dot = lax.dot_general(cb.astype(jnp.bfloat16), zt.astype(jnp.bfloat16),
                          (((1,), (0,)), ((), ())),
                          preferred_element_type=jnp.float32)
    zsqh = jnp.broadcast_to(zsqh_ref[...], (CH, BN))
    acc_v = jnp.full((CH, BN), jnp.inf, jnp.float32)
    acc_c = jnp.zeros((CH, BN), jnp.int32)
    for i in range(NCH):
        d = dot[CH * i:CH * i + CH, :]
        s = zsqh + esqh_ref[pl.ds(CH * i, CH), :]
        dist = s - d
        upd = dist < acc_v
        acc_v = jnp.where(upd, dist, acc_v)
        acc_c = jnp.where(upd, i, acc_c)
    tile_min = jnp.min(acc_v, axis=0, keepdims=True)
    rows = (acc_c * CH + lax.broadcasted_iota(jnp.int32, (CH, BN), 0)
            + ki * BK)
    cand = jnp.where(acc_v == tile_min, rows, BIG)
    tile_arg = jnp.min(cand, axis=0, keepdims=True)

    @pl.when(ki == 0)
    def _():
        minv_ref[...] = tile_min.astype(jnp.bfloat16)
        mini_ref[...] = tile_arg

    @pl.when(ki > 0)
    def _():
        acc = minv_ref[...].astype(jnp.float32)
        better = tile_min < acc
        minv_ref[...] = jnp.where(better, tile_min,
                                  acc).astype(jnp.bfloat16)
        mini_ref[...] = jnp.where(better, tile_arg, mini_ref[...])

    @pl.when(ki == KT - 1)
    def _():
        idx_ref[...] = mini_ref[...]


_argmin_call = pl.pallas_call(
    _argmin_kernel,
    grid=(NT, KT),
    in_specs=[
        pl.BlockSpec((BN, D), lambda ni, ki: (ni, 0)),
        pl.BlockSpec((BK, D), lambda ni, ki: (ki, 0)),
        pl.BlockSpec((1, BN), lambda ni, ki: (0, ni)),
        pl.BlockSpec((BK, 1), lambda ni, ki: (ki, 0)),
    ],
    out_specs=pl.BlockSpec((1, BN), lambda ni, ki: (0, ni)),
    out_shape=jax.ShapeDtypeStruct((1, N_TOK), jnp.int32),
    scratch_shapes=[
        pltpu.VMEM((1, BN), jnp.bfloat16),
        pltpu.VMEM((1, BN), jnp.int32),
        pltpu.VMEM((D, BN), jnp.float32),
    ],
)


def _sc_gather_kernel(cb_hbm, idx_hbm, zq_hbm, idx_v, rows_v, sem):
    wid = lax.axis_index("s") * NC + lax.axis_index("c")
    base = wid * BPW
    pltpu.sync_copy(idx_hbm.at[pl.ds(base, BPW)], idx_v)
    pltpu.async_copy(cb_hbm.at[idx_v], rows_v, sem).wait()
    pltpu.sync_copy(rows_v, zq_hbm.at[pl.ds(base, BPW)])


@functools.lru_cache(maxsize=1)
def _sc_gather_call():
    return pl.kernel(
        _sc_gather_kernel,
        mesh=plsc.VectorSubcoreMesh(core_axis_name="c", subcore_axis_name="s"),
        out_type=jax.ShapeDtypeStruct((N_TOK, D), jnp.float32),
        scratch_types=[
            pltpu.VMEM((BPW,), jnp.int32),
            pltpu.VMEM((BPW, D), jnp.float32),
            pltpu.SemaphoreType.DMA,
        ],
    )


def _epilogue_kernel(z_ref, zq_ref, idx_ref, zqst_ref, loss_ref, perp_ref,
                     acc_ref):
    i = pl.program_id(0)
    z = z_ref[...]
    zq = zq_ref[...]
    zqst_ref[...] = z + (zq - z)
    d = z - zq
    part = jnp.sum(d * d, axis=(0, 1), keepdims=True)

    @pl.when(i == 0)
    def _():
        acc_ref[...] = part

    @pl.when(i > 0)
    def _():
        acc_ref[...] = acc_ref[...] + part

    @pl.when(i == NT - 1)
    def _():
        m = acc_ref[...] * jnp.float32(1.0 / (N_TOK * D))
        loss_ref[...] = m + 0.25 * m
        idx = idx_ref[...]
        hi_bins = lax.broadcasted_iota(jnp.int32, (64, 1), 0)
        lo_bins = lax.broadcasted_iota(jnp.int32, (128, 1), 0)
        oh_hi = jnp.where(lax.shift_right_logical(idx, 7) == hi_bins,
                          1.0, 0.0).astype(jnp.bfloat16)
        oh_lo = jnp.where((idx & 127) == lo_bins,
                          1.0, 0.0).astype(jnp.bfloat16)
        counts = lax.dot_general(oh_hi, oh_lo, (((1,), (1,)), ((), ())),
                                 preferred_element_type=jnp.float32)
        p = counts * jnp.float32(1.0 / N_TOK)
        plogp = p * jnp.log(p + 1e-12)
        perp_ref[...] = jnp.exp(-jnp.sum(plogp, axis=(0, 1), keepdims=True))


_epilogue_call = pl.pallas_call(
    _epilogue_kernel,
    grid=(NT,),
    in_specs=[
        pl.BlockSpec((BN, D), lambda i: (i, 0)),
        pl.BlockSpec((BN, D), lambda i: (i, 0)),
        pl.BlockSpec((1, N_TOK), lambda i: (0, 0)),
    ],
    out_specs=[
        pl.BlockSpec((BN, D), lambda i: (i, 0)),
        pl.BlockSpec((1, 1), lambda i: (0, 0)),
        pl.BlockSpec((1, 1), lambda i: (0, 0)),
    ],
    out_shape=[
        jax.ShapeDtypeStruct((N_TOK, D), jnp.float32),
        jax.ShapeDtypeStruct((1, 1), jnp.float32),
        jax.ShapeDtypeStruct((1, 1), jnp.float32),
    ],
    scratch_shapes=[pltpu.VMEM((1, 1), jnp.float32)],
)


def kernel(z_e, codebook):
    z2d = z_e.reshape(N_TOK, D)
    z_sqh = (jnp.sum(z2d ** 2, axis=1) * 0.5).reshape(1, N_TOK)
    e_sqh = (jnp.sum(codebook ** 2, axis=1) * 0.5).reshape(K_CB, 1)
    idx_row = _argmin_call(z2d, codebook, z_sqh, e_sqh)
    idx_flat = idx_row.reshape(N_TOK)
    zq2d = _sc_gather_call()(codebook, idx_flat)
    zqst2d, loss, perp = _epilogue_call(z2d, zq2d, idx_row)
    return (zqst2d.reshape(z_e.shape),
            idx_flat.reshape(z_e.shape[:-1]),
            loss.reshape(()),
            perp.reshape(()))
